# baseline (speedup 1.0000x reference)
import functools
import jax
import jax.numpy as jnp
from jax import lax
from jax.experimental import pallas as pl
from jax.experimental.pallas import tpu as pltpu
from jax.experimental.pallas import tpu_sc as plsc

_N_BITS = 8
_QN = float(-(2 ** (_N_BITS - 1)) + 1)
_QP = float(2 ** (_N_BITS - 1) - 1)
_XOR = 1 << (_N_BITS - 1)

_NC, _NS, _L = 2, 16, 16
_NW = _NC * _NS
_NR = 4

_MAGIC = 12582912.0


@functools.lru_cache(maxsize=None)
def _build(nrow, ncol, n_flip):
  rows_per_tile = nrow // _NW
  nchunk = rows_per_tile // _NR
  chunk_elems = _NR * ncol
  n_fvec = -(-n_flip // _L)
  col_shift = ncol.bit_length() - 1
  assert 1 << col_shift == ncol
  mesh = plsc.VectorSubcoreMesh(
      core_axis_name="core", subcore_axis_name="subcore",
      num_cores=_NC, num_subcores=_NS)

  @functools.partial(
      pl.kernel,
      out_type=jax.ShapeDtypeStruct((nrow, ncol), jnp.float32),
      mesh=mesh,
      compiler_params=pltpu.CompilerParams(needs_layout_passes=False),
      scratch_types=[
          pltpu.VMEM((_NR, ncol), jnp.float32),
          pltpu.VMEM((_NR, ncol), jnp.float32),
          pltpu.VMEM((_NR, ncol), jnp.float32),
          pltpu.VMEM((_NR, ncol), jnp.float32),
          pltpu.VMEM((n_flip,), jnp.int32),
          pltpu.VMEM((n_flip + _L,), jnp.int32),
          pltpu.VMEM((_L,), jnp.float32),
          pltpu.SemaphoreType.DMA,
          pltpu.SemaphoreType.DMA,
          pltpu.SemaphoreType.DMA,
          pltpu.SemaphoreType.DMA,
      ],
  )
  def launch(w_hbm, alpha_hbm, fidx_hbm, out_hbm,
             in0, in1, o0, o1, idx_v, tidx_v, alpha_ref,
             isem0, isem1, osem0, osem1):
    wid = lax.axis_index("subcore") * _NC + lax.axis_index("core")
    row_t = wid * rows_per_tile
    base_t = row_t * ncol

    ins = (in0, in1)
    outs = (o0, o1)
    isems = (isem0, isem1)
    osems = (osem0, osem1)
    in_d = [None] * nchunk
    out_d = [None] * nchunk

    in_d[0] = pltpu.async_copy(
        w_hbm.at[pl.ds(row_t, _NR), :], ins[0], isems[0])

    pltpu.sync_copy(fidx_hbm, idx_v)
    pltpu.sync_copy(alpha_hbm, alpha_ref)
    lane = lax.iota(jnp.int32, _L)
    alpha_v = jnp.maximum(alpha_ref[...], 1e-4)
    inv_v = 1.0 / alpha_v

    def _compact(j, cnt):
      start = jnp.maximum(jnp.minimum(j * _L, n_flip - _L), 0)
      gid = start + lane
      iv = idx_v[pl.ds(start, _L)]
      m = ((gid >= j * _L)
           & (iv >= base_t) & (iv < base_t + rows_per_tile * ncol))
      plsc.store_compressed(tidx_v.at[pl.ds(cnt, _L)], iv - base_t, mask=m)
      return cnt + jnp.sum(m.astype(jnp.int32))

    cnt = lax.fori_loop(0, n_fvec, _compact, jnp.int32(0))
    n_tvec = (cnt + _L - 1) // _L

    for c in range(nchunk):
      cur = c & 1
      row = row_t + c * _NR
      if c + 1 < nchunk:
        in_d[c + 1] = pltpu.async_copy(
            w_hbm.at[pl.ds(row + _NR, _NR), :],
            ins[(c + 1) & 1], isems[(c + 1) & 1])
      in_d[c].wait()
      if c >= 2:
        out_d[c - 2].wait()

      in_ref = ins[cur]
      out_ref = outs[cur]

      @plsc.parallel_loop(0, _NR, step=1)
      def _rows(rr):
        @plsc.parallel_loop(0, ncol, step=_L, unroll=8)
        def _dense(i):
          x = in_ref[rr, pl.ds(i, _L)]
          q = jnp.minimum(jnp.maximum(x * inv_v, _QN), _QP)
          r = (q + _MAGIC) - _MAGIC
          out_ref[rr, pl.ds(i, _L)] = r * alpha_v

      cbase = c * chunk_elems

      def _flips(j, _):
        lv = tidx_v[pl.ds(j * _L, _L)]
        m = ((j * _L + lane < cnt)
             & (lv >= cbase) & (lv < cbase + chunk_elems))
        loc = jnp.minimum(jnp.maximum(lv - cbase, 0), chunk_elems - 1)
        loc_r = lax.shift_right_logical(loc, col_shift)
        loc_c = loc & (ncol - 1)
        wv = plsc.load_gather(in_ref, [loc_r, loc_c], mask=m)
        q = jnp.minimum(jnp.maximum(wv * inv_v, _QN), _QP)
        t = q.astype(jnp.int32) ^ _XOR
        plsc.store_scatter(out_ref, [loc_r, loc_c],
                           t.astype(jnp.float32) * alpha_v, mask=m)
        return 0

      lax.fori_loop(0, n_tvec, _flips, 0)

      out_d[c] = pltpu.async_copy(
          out_ref, out_hbm.at[pl.ds(row, _NR), :], osems[cur])

    out_d[nchunk - 2].wait()
    out_d[nchunk - 1].wait()

  return launch


def kernel(weight, alpha, flip_idx):
  nrow, ncol = weight.shape
  alpha16 = jnp.broadcast_to(
      alpha.astype(jnp.float32).reshape(()), (_L,))
  return _build(nrow, ncol, flip_idx.shape[0])(
      weight, alpha16, flip_idx.astype(jnp.int32))


# speedup vs baseline: 1.0914x; 1.0914x over previous
import functools
import jax
import jax.numpy as jnp
from jax import lax
from jax.experimental import pallas as pl
from jax.experimental.pallas import tpu as pltpu
from jax.experimental.pallas import tpu_sc as plsc

_N_BITS = 8
_QN = float(-(2 ** (_N_BITS - 1)) + 1)
_QP = float(2 ** (_N_BITS - 1) - 1)
_XOR = 1 << (_N_BITS - 1)

_NC, _NS, _L = 2, 16, 16
_NW = _NC * _NS
_NR = 4

_MAGIC = 12582912.0


@functools.lru_cache(maxsize=None)
def _build(nrow, ncol, n_flip):
  rows_per_tile = nrow // _NW
  nchunk = rows_per_tile // _NR
  chunk_elems = _NR * ncol
  n_fvec = -(-n_flip // _L)
  col_shift = ncol.bit_length() - 1
  assert 1 << col_shift == ncol
  mesh = plsc.VectorSubcoreMesh(
      core_axis_name="core", subcore_axis_name="subcore",
      num_cores=_NC, num_subcores=_NS)

  @functools.partial(
      pl.kernel,
      out_type=jax.ShapeDtypeStruct((nrow, ncol), jnp.float32),
      mesh=mesh,
      compiler_params=pltpu.CompilerParams(needs_layout_passes=False),
      scratch_types=[
          pltpu.VMEM((_NR, ncol), jnp.float32),
          pltpu.VMEM((_NR, ncol), jnp.float32),
          pltpu.VMEM((_NR, ncol), jnp.float32),
          pltpu.VMEM((_NR, ncol), jnp.float32),
          pltpu.VMEM((_NR, ncol), jnp.float32),
          pltpu.VMEM((_NR, ncol), jnp.float32),
          pltpu.VMEM((n_flip,), jnp.int32),
          pltpu.VMEM((n_flip + _L,), jnp.int32),
          pltpu.VMEM((_L,), jnp.float32),
          pltpu.SemaphoreType.DMA,
          pltpu.SemaphoreType.DMA,
          pltpu.SemaphoreType.DMA,
          pltpu.SemaphoreType.DMA,
          pltpu.SemaphoreType.DMA,
          pltpu.SemaphoreType.DMA,
      ],
  )
  def launch(w_hbm, alpha_hbm, fidx_hbm, out_hbm,
             in0, in1, in2, o0, o1, o2, idx_v, tidx_v, alpha_ref,
             isem0, isem1, isem2, osem0, osem1, osem2):
    wid = lax.axis_index("subcore") * _NC + lax.axis_index("core")
    row_t = wid * rows_per_tile
    base_t = row_t * ncol

    ins = (in0, in1, in2)
    outs = (o0, o1, o2)
    isems = (isem0, isem1, isem2)
    osems = (osem0, osem1, osem2)
    in_d = [None] * nchunk
    out_d = [None] * nchunk

    for c0 in range(2):
      in_d[c0] = pltpu.async_copy(
          w_hbm.at[pl.ds(row_t + c0 * _NR, _NR), :], ins[c0], isems[c0])

    pltpu.sync_copy(fidx_hbm, idx_v)
    pltpu.sync_copy(alpha_hbm, alpha_ref)
    lane = lax.iota(jnp.int32, _L)
    alpha_v = jnp.maximum(alpha_ref[...], 1e-4)
    inv_v = 1.0 / alpha_v

    def _compact(j, cnt):
      start = jnp.maximum(jnp.minimum(j * _L, n_flip - _L), 0)
      gid = start + lane
      iv = idx_v[pl.ds(start, _L)]
      m = ((gid >= j * _L)
           & (iv >= base_t) & (iv < base_t + rows_per_tile * ncol))
      plsc.store_compressed(tidx_v.at[pl.ds(cnt, _L)], iv - base_t, mask=m)
      return cnt + jnp.sum(m.astype(jnp.int32))

    cnt = lax.fori_loop(0, n_fvec, _compact, jnp.int32(0))
    n_tvec = (cnt + _L - 1) // _L

    for c in range(nchunk):
      cur = c % 3
      row = row_t + c * _NR
      if c + 2 < nchunk:
        in_d[c + 2] = pltpu.async_copy(
            w_hbm.at[pl.ds(row + 2 * _NR, _NR), :],
            ins[(c + 2) % 3], isems[(c + 2) % 3])
      in_d[c].wait()
      if c >= 3:
        out_d[c - 3].wait()

      in_ref = ins[cur]
      out_ref = outs[cur]

      @plsc.parallel_loop(0, _NR, step=1)
      def _rows(rr):
        @plsc.parallel_loop(0, ncol, step=_L, unroll=8)
        def _dense(i):
          x = in_ref[rr, pl.ds(i, _L)]
          q = jnp.minimum(jnp.maximum(x * inv_v, _QN), _QP)
          r = (q + _MAGIC) - _MAGIC
          out_ref[rr, pl.ds(i, _L)] = r * alpha_v

      cbase = c * chunk_elems

      def _flips(j, _):
        lv = tidx_v[pl.ds(j * _L, _L)]
        m = ((j * _L + lane < cnt)
             & (lv >= cbase) & (lv < cbase + chunk_elems))
        loc = jnp.minimum(jnp.maximum(lv - cbase, 0), chunk_elems - 1)
        loc_r = lax.shift_right_logical(loc, col_shift)
        loc_c = loc & (ncol - 1)
        wv = plsc.load_gather(in_ref, [loc_r, loc_c], mask=m)
        q = jnp.minimum(jnp.maximum(wv * inv_v, _QN), _QP)
        t = q.astype(jnp.int32) ^ _XOR
        plsc.store_scatter(out_ref, [loc_r, loc_c],
                           t.astype(jnp.float32) * alpha_v, mask=m)
        return 0

      lax.fori_loop(0, n_tvec, _flips, 0)

      out_d[c] = pltpu.async_copy(
          out_ref, out_hbm.at[pl.ds(row, _NR), :], osems[cur])

    out_d[nchunk - 3].wait()
    out_d[nchunk - 2].wait()
    out_d[nchunk - 1].wait()

  return launch


def kernel(weight, alpha, flip_idx):
  nrow, ncol = weight.shape
  alpha16 = jnp.broadcast_to(
      alpha.astype(jnp.float32).reshape(()), (_L,))
  return _build(nrow, ncol, flip_idx.shape[0])(
      weight, alpha16, flip_idx.astype(jnp.int32))


# speedup vs baseline: 1.0924x; 1.0009x over previous
import functools
import jax
import jax.numpy as jnp
from jax import lax
from jax.experimental import pallas as pl
from jax.experimental.pallas import tpu as pltpu
from jax.experimental.pallas import tpu_sc as plsc

_N_BITS = 8
_QN = float(-(2 ** (_N_BITS - 1)) + 1)
_QP = float(2 ** (_N_BITS - 1) - 1)
_XOR = 1 << (_N_BITS - 1)

_NC, _NS, _L = 2, 16, 16
_NW = _NC * _NS
_NR = 8
_HC = 2

_MAGIC = 12582912.0


@functools.lru_cache(maxsize=None)
def _build(nrow, ncol, n_flip):
  rows_per_tile = nrow // _NW
  ccol = ncol // _HC
  nband = rows_per_tile // _NR
  nchunk = nband * _HC
  n_fvec = -(-n_flip // _L)
  col_shift = ncol.bit_length() - 1
  assert 1 << col_shift == ncol
  ccol_shift = ccol.bit_length() - 1
  assert 1 << ccol_shift == ccol
  nr_shift = _NR.bit_length() - 1
  mesh = plsc.VectorSubcoreMesh(
      core_axis_name="core", subcore_axis_name="subcore",
      num_cores=_NC, num_subcores=_NS)

  @functools.partial(
      pl.kernel,
      out_type=jax.ShapeDtypeStruct((nrow, ncol), jnp.float32),
      mesh=mesh,
      compiler_params=pltpu.CompilerParams(needs_layout_passes=False),
      scratch_types=[
          pltpu.VMEM((_NR, ccol), jnp.float32),
          pltpu.VMEM((_NR, ccol), jnp.float32),
          pltpu.VMEM((_NR, ccol), jnp.float32),
          pltpu.VMEM((_NR, ccol), jnp.float32),
          pltpu.VMEM((_NR, ccol), jnp.float32),
          pltpu.VMEM((_NR, ccol), jnp.float32),
          pltpu.VMEM((n_flip,), jnp.int32),
          pltpu.VMEM((n_flip + _L,), jnp.int32),
          pltpu.VMEM((_L,), jnp.float32),
          pltpu.SemaphoreType.DMA,
          pltpu.SemaphoreType.DMA,
          pltpu.SemaphoreType.DMA,
          pltpu.SemaphoreType.DMA,
          pltpu.SemaphoreType.DMA,
          pltpu.SemaphoreType.DMA,
      ],
  )
  def launch(w_hbm, alpha_hbm, fidx_hbm, out_hbm,
             in0, in1, in2, o0, o1, o2, idx_v, tidx_v, alpha_ref,
             isem0, isem1, isem2, osem0, osem1, osem2):
    wid = lax.axis_index("subcore") * _NC + lax.axis_index("core")
    row_t = wid * rows_per_tile
    base_t = row_t * ncol

    ins = (in0, in1, in2)
    outs = (o0, o1, o2)
    isems = (isem0, isem1, isem2)
    osems = (osem0, osem1, osem2)
    in_d = [None] * nchunk
    out_d = [None] * nchunk

    def _chunk_slice(ref, c):
      row0 = row_t + (c // _HC) * _NR
      col0 = (c % _HC) * ccol
      return ref.at[pl.ds(row0, _NR), pl.ds(col0, ccol)]

    for c0 in range(2):
      in_d[c0] = pltpu.async_copy(
          _chunk_slice(w_hbm, c0), ins[c0], isems[c0])

    pltpu.sync_copy(fidx_hbm, idx_v)
    pltpu.sync_copy(alpha_hbm, alpha_ref)
    lane = lax.iota(jnp.int32, _L)
    alpha_v = jnp.maximum(alpha_ref[...], 1e-4)
    inv_v = 1.0 / alpha_v

    def _compact(j, cnt):
      start = jnp.maximum(jnp.minimum(j * _L, n_flip - _L), 0)
      gid = start + lane
      iv = idx_v[pl.ds(start, _L)]
      m = ((gid >= j * _L)
           & (iv >= base_t) & (iv < base_t + rows_per_tile * ncol))
      plsc.store_compressed(tidx_v.at[pl.ds(cnt, _L)], iv - base_t, mask=m)
      return cnt + jnp.sum(m.astype(jnp.int32))

    cnt = lax.fori_loop(0, n_fvec, _compact, jnp.int32(0))
    n_tvec = (cnt + _L - 1) // _L

    for c in range(nchunk):
      cur = c % 3
      if c + 2 < nchunk:
        in_d[c + 2] = pltpu.async_copy(
            _chunk_slice(w_hbm, c + 2), ins[(c + 2) % 3], isems[(c + 2) % 3])
      in_d[c].wait()
      if c >= 3:
        out_d[c - 3].wait()

      in_ref = ins[cur]
      out_ref = outs[cur]

      @plsc.parallel_loop(0, _NR, step=1)
      def _rows(rr):
        @plsc.parallel_loop(0, ccol, step=_L, unroll=8)
        def _dense(i):
          x = in_ref[rr, pl.ds(i, _L)]
          q = jnp.minimum(jnp.maximum(x * inv_v, _QN), _QP)
          r = (q + _MAGIC) - _MAGIC
          out_ref[rr, pl.ds(i, _L)] = r * alpha_v

      band = c // _HC
      half = c % _HC

      def _flips(j, _):
        lv = tidx_v[pl.ds(j * _L, _L)]
        band_of = lax.shift_right_logical(lv, col_shift + nr_shift)
        col = lv & (ncol - 1)
        half_of = lax.shift_right_logical(col, ccol_shift)
        m = ((j * _L + lane < cnt)
             & (band_of == band) & (half_of == half))
        br = lax.shift_right_logical(lv, col_shift) & (_NR - 1)
        bc = col & (ccol - 1)
        wv = plsc.load_gather(in_ref, [br, bc], mask=m)
        q = jnp.minimum(jnp.maximum(wv * inv_v, _QN), _QP)
        t = q.astype(jnp.int32) ^ _XOR
        plsc.store_scatter(out_ref, [br, bc],
                           t.astype(jnp.float32) * alpha_v, mask=m)
        return 0

      lax.fori_loop(0, n_tvec, _flips, 0)

      out_d[c] = pltpu.async_copy(
          out_ref, _chunk_slice(out_hbm, c), osems[cur])

    out_d[nchunk - 3].wait()
    out_d[nchunk - 2].wait()
    out_d[nchunk - 1].wait()

  return launch


def kernel(weight, alpha, flip_idx):
  nrow, ncol = weight.shape
  alpha16 = jnp.broadcast_to(
      alpha.astype(jnp.float32).reshape(()), (_L,))
  return _build(nrow, ncol, flip_idx.shape[0])(
      weight, alpha16, flip_idx.astype(jnp.int32))


# speedup vs baseline: 1.2147x; 1.1120x over previous
import functools
import jax
import jax.numpy as jnp
from jax import lax
from jax.experimental import pallas as pl
from jax.experimental.pallas import tpu as pltpu
from jax.experimental.pallas import tpu_sc as plsc

_N_BITS = 8
_QN = float(-(2 ** (_N_BITS - 1)) + 1)
_QP = float(2 ** (_N_BITS - 1) - 1)
_XOR = 1 << (_N_BITS - 1)

_NC, _NS, _L = 2, 16, 16
_NW = _NC * _NS
_NR = 8
_HC = 2

_MAGIC = 12582912.0


@functools.lru_cache(maxsize=None)
def _build(nrow, ncol, n_flip):
  rows_per_tile = nrow // _NW
  ccol = ncol // _HC
  nband = rows_per_tile // _NR
  nchunk = nband * _HC
  n_fvec = -(-n_flip // _L)
  col_shift = ncol.bit_length() - 1
  assert 1 << col_shift == ncol
  ccol_shift = ccol.bit_length() - 1
  assert 1 << ccol_shift == ccol
  nr_shift = _NR.bit_length() - 1
  mesh = plsc.VectorSubcoreMesh(
      core_axis_name="core", subcore_axis_name="subcore",
      num_cores=_NC, num_subcores=_NS)

  @functools.partial(
      pl.kernel,
      out_type=jax.ShapeDtypeStruct((nrow, ncol), jnp.float32),
      mesh=mesh,
      compiler_params=pltpu.CompilerParams(needs_layout_passes=False),
      scratch_types=[
          pltpu.VMEM((_NR, ccol), jnp.float32),
          pltpu.VMEM((_NR, ccol), jnp.float32),
          pltpu.VMEM((_NR, ccol), jnp.float32),
          pltpu.VMEM((_NR, ccol), jnp.float32),
          pltpu.VMEM((_NR, ccol), jnp.float32),
          pltpu.VMEM((_NR, ccol), jnp.float32),
          pltpu.VMEM((n_flip,), jnp.int32),
          pltpu.VMEM((n_flip + _L,), jnp.int32),
          pltpu.VMEM((_L,), jnp.float32),
          pltpu.SemaphoreType.DMA,
          pltpu.SemaphoreType.DMA,
          pltpu.SemaphoreType.DMA,
          pltpu.SemaphoreType.DMA,
          pltpu.SemaphoreType.DMA,
          pltpu.SemaphoreType.DMA,
      ],
  )
  def launch(w_hbm, alpha_hbm, fidx_hbm, out_hbm,
             in0, in1, in2, o0, o1, o2, idx_v, tidx_v, alpha_ref,
             isem0, isem1, isem2, osem0, osem1, osem2):
    wid = lax.axis_index("subcore") * _NC + lax.axis_index("core")
    row_t = wid * rows_per_tile
    base_t = row_t * ncol

    ins = (in0, in1, in2)
    outs = (o0, o1, o2)
    isems = (isem0, isem1, isem2)
    osems = (osem0, osem1, osem2)

    def _chunk_slice(ref, c):
      if isinstance(c, int):
        band, half = c // _HC, c % _HC
      else:
        band = lax.shift_right_logical(c, 1)
        half = c & 1
      row0 = row_t + band * _NR
      col0 = half * ccol
      return ref.at[pl.ds(row0, _NR), pl.ds(col0, ccol)]

    for c0 in range(2):
      pltpu.async_copy(_chunk_slice(w_hbm, c0), ins[c0], isems[c0])

    pltpu.sync_copy(fidx_hbm, idx_v)
    pltpu.sync_copy(alpha_hbm, alpha_ref)
    lane = lax.iota(jnp.int32, _L)
    alpha_v = jnp.maximum(alpha_ref[...], 1e-4)
    inv_v = 1.0 / alpha_v

    def _compact(j, cnt):
      start = jnp.maximum(jnp.minimum(j * _L, n_flip - _L), 0)
      gid = start + lane
      iv = idx_v[pl.ds(start, _L)]
      m = ((gid >= j * _L)
           & (iv >= base_t) & (iv < base_t + rows_per_tile * ncol))
      plsc.store_compressed(tidx_v.at[pl.ds(cnt, _L)], iv - base_t, mask=m)
      return cnt + jnp.sum(m.astype(jnp.int32))

    cnt = lax.fori_loop(0, n_fvec, _compact, jnp.int32(0))
    n_tvec = (cnt + _L - 1) // _L

    def _do_chunk(ci, k, prefetch, wait_out):
      if prefetch:
        pltpu.async_copy(_chunk_slice(w_hbm, ci + 2),
                         ins[(k + 2) % 3], isems[(k + 2) % 3])
      pltpu.make_async_copy(_chunk_slice(w_hbm, ci), ins[k], isems[k]).wait()

      def _wait_out():
        pltpu.make_async_copy(
            outs[k], _chunk_slice(out_hbm, ci), osems[k]).wait()

      if wait_out is True:
        _wait_out()
      elif wait_out is not False:
        pl.when(wait_out)(_wait_out)

      in_ref = ins[k]
      out_ref = outs[k]

      @plsc.parallel_loop(0, _NR, step=1)
      def _rows(rr):
        @plsc.parallel_loop(0, ccol, step=_L, unroll=8)
        def _dense(i):
          x = in_ref[rr, pl.ds(i, _L)]
          q = jnp.minimum(jnp.maximum(x * inv_v, _QN), _QP)
          r = (q + _MAGIC) - _MAGIC
          out_ref[rr, pl.ds(i, _L)] = r * alpha_v

      if isinstance(ci, int):
        band, half = ci // _HC, ci % _HC
      else:
        band = lax.shift_right_logical(ci, 1)
        half = ci & 1

      def _flips(j, _):
        lv = tidx_v[pl.ds(j * _L, _L)]
        band_of = lax.shift_right_logical(lv, col_shift + nr_shift)
        col = lv & (ncol - 1)
        half_of = lax.shift_right_logical(col, ccol_shift)
        m = ((j * _L + lane < cnt)
             & (band_of == band) & (half_of == half))
        br = lax.shift_right_logical(lv, col_shift) & (_NR - 1)
        bc = col & (ccol - 1)
        wv = plsc.load_gather(in_ref, [br, bc], mask=m)
        q = jnp.minimum(jnp.maximum(wv * inv_v, _QN), _QP)
        t = q.astype(jnp.int32) ^ _XOR
        plsc.store_scatter(out_ref, [br, bc],
                           t.astype(jnp.float32) * alpha_v, mask=m)
        return 0

      lax.fori_loop(0, n_tvec, _flips, 0)

      pltpu.async_copy(out_ref, _chunk_slice(out_hbm, ci), osems[k])

    n_steady = nchunk - 2
    assert n_steady % 3 == 0 and _HC == 2

    def _round_body(r, _):
      for k in range(3):
        _do_chunk(3 * r + k, k, prefetch=True, wait_out=r > 0)
      return 0

    lax.fori_loop(0, n_steady // 3, _round_body, 0)

    for c in (nchunk - 2, nchunk - 1):
      _do_chunk(c, c % 3, prefetch=False, wait_out=True)

    for c in (nchunk - 3, nchunk - 2, nchunk - 1):
      pltpu.make_async_copy(
          outs[c % 3], _chunk_slice(out_hbm, c), osems[c % 3]).wait()

  return launch


def kernel(weight, alpha, flip_idx):
  nrow, ncol = weight.shape
  alpha16 = jnp.broadcast_to(
      alpha.astype(jnp.float32).reshape(()), (_L,))
  return _build(nrow, ncol, flip_idx.shape[0])(
      weight, alpha16, flip_idx.astype(jnp.int32))
